# baseline (speedup 1.0000x reference)
import jax
import jax.numpy as jnp
from jax import lax
from jax.experimental import pallas as pl
from jax.experimental.pallas import tpu as pltpu
from jax.experimental.pallas import tpu_sc as plsc

N = 10000
E = 320000
H = 128
EMB = 64

NS = 16
CH = 128
NCHUNK = E // CH
NPAIRS = 78
N_PAD = 10240
ROWS_PER_TILE = N_PAD // NS
WCH = 128
CNTW = 16


def _zero_fill(buf, nrows, width, value=0.0):
  def zrow(r, carry):
    for j in range(width // 16):
      buf[r, pl.ds(16 * j, 16)] = jnp.full((16,), value, jnp.float32)
    return carry
  lax.fori_loop(0, nrows, zrow, 0)


def _sc_agg_body(hs_hbm, ha_hbm, sd_sa, sd_as,
                 out_sa, out_as, acc, isd, rows0, rows1,
                 isem, gsem0, gsem1, ssem0, ssem1):
  c = lax.axis_index("c")
  s = lax.axis_index("s")
  r0row = 314 * s

  _zero_fill(rows0, WCH, H)
  base = s * ROWS_PER_TILE
  for k in range(ROWS_PER_TILE // WCH):
    pltpu.sync_copy(rows0, acc.at[pl.ds(base + k * WCH, WCH)])
  plsc.subcore_barrier()

  def run(table, sd_hbm):
    pltpu.async_copy(sd_hbm.at[pl.ds(r0row, 4)], isd.at[pl.ds(0, 4)], isem)

    def pair(q, carry):
      p4 = 4 * (q & 1)
      pn4 = 4 - p4
      sa_r = p4
      da_r = p4 + 1
      sb_r = p4 + 2
      db_r = p4 + 3
      pltpu.make_async_copy(sd_hbm.at[pl.ds(r0row, 4)], isd.at[pl.ds(0, 4)],
                            isem).wait()

      @pl.when(q > 0)
      def _():
        pltpu.make_async_copy(rows0, acc.at[isd.at[da_r]], ssem0).wait()
      pltpu.async_copy(table.at[isd.at[sa_r]], rows0, gsem0).wait()
      pltpu.async_copy(rows0, acc.at[isd.at[da_r]], ssem0, add=True)

      @pl.when(q > 0)
      def _():
        pltpu.make_async_copy(rows1, acc.at[isd.at[db_r]], ssem1).wait()
      pltpu.async_copy(sd_hbm.at[pl.ds(r0row + 4 * (q + 1), 4)],
                       isd.at[pl.ds(pn4, 4)], isem)
      pltpu.async_copy(table.at[isd.at[sb_r]], rows1, gsem1).wait()
      pltpu.async_copy(rows1, acc.at[isd.at[db_r]], ssem1, add=True)
      return carry

    lax.fori_loop(0, NPAIRS, pair, 0)

    pltpu.make_async_copy(sd_hbm.at[pl.ds(r0row, 4)], isd.at[pl.ds(0, 4)],
                          isem).wait()
    pltpu.make_async_copy(rows0, acc.at[isd.at[1]], ssem0).wait()
    pltpu.make_async_copy(rows1, acc.at[isd.at[1]], ssem1).wait()

    pltpu.async_copy(table.at[isd.at[0]], rows0, gsem0).wait()
    pltpu.sync_copy(rows0, acc.at[isd.at[1]], add=True)

  @pl.when(c == 0)
  def _():
    run(hs_hbm, sd_sa)

  @pl.when(c == 1)
  def _():
    run(ha_hbm, sd_as)

  plsc.subcore_barrier()

  def writeout(out_hbm):
    pltpu.sync_copy(acc.at[pl.ds(base, ROWS_PER_TILE)],
                    out_hbm.at[pl.ds(base, ROWS_PER_TILE)])

  @pl.when(c == 0)
  def _():
    writeout(out_sa)

  @pl.when(c == 1)
  def _():
    writeout(out_as)


_sc_agg = pl.kernel(
    _sc_agg_body,
    out_type=(jax.ShapeDtypeStruct((N_PAD, H), jnp.float32),
              jax.ShapeDtypeStruct((N_PAD, H), jnp.float32)),
    mesh=plsc.VectorSubcoreMesh(core_axis_name="c", subcore_axis_name="s"),
    compiler_params=pltpu.CompilerParams(use_tc_tiling_on_sc=False),
    scratch_types=[
        pltpu.VMEM_SHARED((N_PAD, H), jnp.float32),
        pltpu.VMEM((8, CH), jnp.int32),
        pltpu.VMEM((CH, H), jnp.float32),
        pltpu.VMEM((CH, H), jnp.float32),
        pltpu.SemaphoreType.DMA,
        pltpu.SemaphoreType.DMA,
        pltpu.SemaphoreType.DMA,
        pltpu.SemaphoreType.DMA,
        pltpu.SemaphoreType.DMA,
    ],
)


def _sc_cnt_body(dst_sa, dst_as, out_sa, out_as, acc, idst, ones, sem):
  c = lax.axis_index("c")
  s = lax.axis_index("s")
  q0 = 156 * s + jnp.minimum(s, 4)
  n_t = jnp.where(s < 4, 157, 156)
  q0c = jnp.minimum(q0, NCHUNK - 157)
  joff = q0 - q0c

  _zero_fill(ones, WCH, CNTW)
  base = s * ROWS_PER_TILE
  for k in range(ROWS_PER_TILE // WCH):
    pltpu.sync_copy(ones, acc.at[pl.ds(base + k * WCH, WCH)])
  plsc.subcore_barrier()
  _zero_fill(ones, WCH, CNTW, 1.0)

  def run(dst_hbm):
    pltpu.sync_copy(dst_hbm.at[pl.ds(q0c, 157)], idst)

    def chunk(j, carry):
      pltpu.sync_copy(ones, acc.at[idst.at[j + joff]], add=True)
      return carry
    lax.fori_loop(0, n_t, chunk, 0)

  @pl.when(c == 0)
  def _():
    run(dst_sa)

  @pl.when(c == 1)
  def _():
    run(dst_as)

  plsc.subcore_barrier()

  def writeout(out_hbm):
    pltpu.sync_copy(acc.at[pl.ds(base, ROWS_PER_TILE)],
                    out_hbm.at[pl.ds(base, ROWS_PER_TILE)])

  @pl.when(c == 0)
  def _():
    writeout(out_sa)

  @pl.when(c == 1)
  def _():
    writeout(out_as)


_sc_cnt = pl.kernel(
    _sc_cnt_body,
    out_type=(jax.ShapeDtypeStruct((N_PAD, CNTW), jnp.float32),
              jax.ShapeDtypeStruct((N_PAD, CNTW), jnp.float32)),
    mesh=plsc.VectorSubcoreMesh(core_axis_name="c", subcore_axis_name="s"),
    compiler_params=pltpu.CompilerParams(use_tc_tiling_on_sc=False),
    scratch_types=[
        pltpu.VMEM_SHARED((N_PAD, CNTW), jnp.float32),
        pltpu.VMEM((157, CH), jnp.int32),
        pltpu.VMEM((WCH, CNTW), jnp.float32),
        pltpu.SemaphoreType.DMA,
    ],
)


RBLK = 1000
GRID = N // RBLK


def _ln(x, gamma, beta):
  mu = jnp.mean(x, axis=-1, keepdims=True)
  var = jnp.mean((x - mu) ** 2, axis=-1, keepdims=True)
  return (x - mu) * lax.rsqrt(var + 1e-5) * gamma + beta


def _row_spec(d):
  return pl.BlockSpec((RBLK, d), lambda i: (i, 0))


def _full_spec(a, b):
  return pl.BlockSpec((a, b), lambda i: (0, 0))


def _kin_body(xs, xa, Wls, bls, Wla, bla, hs_out, ha_out):
  hs_out[...] = jnp.dot(xs[...], Wls[...],
                        preferred_element_type=jnp.float32) + bls[...]
  ha_out[...] = jnp.dot(xa[...], Wla[...],
                        preferred_element_type=jnp.float32) + bla[...]


def _kin(xs, xa, Wls, bls, Wla, bla):
  return pl.pallas_call(
      _kin_body,
      grid=(GRID,),
      in_specs=[_row_spec(H), _row_spec(H), _full_spec(H, H), _full_spec(1, H),
                _full_spec(H, H), _full_spec(1, H)],
      out_specs=(_row_spec(H), _row_spec(H)),
      out_shape=(jax.ShapeDtypeStruct((N, H), jnp.float32),
                 jax.ShapeDtypeStruct((N, H), jnp.float32)),
  )(xs, xa, Wls, bls, Wla, bla)


def _upd_one(s_agg, inv, h_dst, Wnr, b, ln_g, ln_b):
  cat = jnp.concatenate([s_agg * inv, h_dst], axis=1)
  new = jnp.dot(cat, Wnr[...], preferred_element_type=jnp.float32) + b[...]
  return _ln(jnp.maximum(new, 0.0), ln_g[...], ln_b[...]) + h_dst


def _kupd_body(ssa, sas, hs, ha, ca, cs, Wnr_sa, b_sa,
               Wnr_as, b_as, lgs, lbs, lga, lba, hs_o, ha_o):
  inv_a = 1.0 / jnp.maximum(ca[:, 0:1], 1.0)
  inv_s = 1.0 / jnp.maximum(cs[:, 0:1], 1.0)
  ha_o[...] = _upd_one(ssa[...], inv_a, ha[...], Wnr_sa, b_sa, lga, lba)
  hs_o[...] = _upd_one(sas[...], inv_s, hs[...], Wnr_as, b_as, lgs, lbs)


def _kupd(ssa, sas, hs, ha, ca, cs, Wnr_sa, b_sa,
          Wnr_as, b_as, lgs, lbs, lga, lba):
  wspec = _full_spec(2 * H, H)
  vspec = _full_spec(1, H)
  return pl.pallas_call(
      _kupd_body,
      grid=(GRID,),
      in_specs=[_row_spec(H), _row_spec(H), _row_spec(H), _row_spec(H),
                _row_spec(CNTW), _row_spec(CNTW),
                wspec, vspec, wspec, vspec,
                vspec, vspec, vspec, vspec],
      out_specs=(_row_spec(H), _row_spec(H)),
      out_shape=(jax.ShapeDtypeStruct((N, H), jnp.float32),
                 jax.ShapeDtypeStruct((N, H), jnp.float32)),
  )(ssa, sas, hs, ha, ca, cs, Wnr_sa, b_sa,
    Wnr_as, b_as, lgs, lbs, lga, lba)


def _out_proj(h, Wo, bo, g):
  o = jnp.dot(h, Wo[...], preferred_element_type=jnp.float32) + bo[...]
  nrm = jnp.sqrt(jnp.sum(o * o, axis=-1, keepdims=True))
  return o / jnp.maximum(nrm, 1e-12) * g[...]


def _kupd_out_body(ssa, sas, hs, ha, ca, cs, Wnr_sa, b_sa,
                   Wnr_as, b_as, lgs, lbs, lga, lba, Wo, bo, g,
                   os_o, oa_o):
  inv_a = 1.0 / jnp.maximum(ca[:, 0:1], 1.0)
  inv_s = 1.0 / jnp.maximum(cs[:, 0:1], 1.0)
  ha2 = _upd_one(ssa[...], inv_a, ha[...], Wnr_sa, b_sa, lga, lba)
  hs2 = _upd_one(sas[...], inv_s, hs[...], Wnr_as, b_as, lgs, lbs)
  os_o[...] = _out_proj(hs2, Wo, bo, g)
  oa_o[...] = _out_proj(ha2, Wo, bo, g)


def _kupd_out(ssa, sas, hs, ha, ca, cs, Wnr_sa, b_sa,
              Wnr_as, b_as, lgs, lbs, lga, lba, Wo, bo, g):
  wspec = _full_spec(2 * H, H)
  vspec = _full_spec(1, H)
  return pl.pallas_call(
      _kupd_out_body,
      grid=(GRID,),
      in_specs=[_row_spec(H), _row_spec(H), _row_spec(H), _row_spec(H),
                _row_spec(CNTW), _row_spec(CNTW),
                wspec, vspec, wspec, vspec,
                vspec, vspec, vspec, vspec,
                _full_spec(H, EMB), _full_spec(1, EMB), _full_spec(1, EMB)],
      out_specs=(_row_spec(EMB), _row_spec(EMB)),
      out_shape=(jax.ShapeDtypeStruct((N, EMB), jnp.float32),
                 jax.ShapeDtypeStruct((N, EMB), jnp.float32)),
  )(ssa, sas, hs, ha, ca, cs, Wnr_sa, b_sa,
    Wnr_as, b_as, lgs, lbs, lga, lba, Wo, bo, g)


def kernel(x_source, x_agent, edge_index_sa, edge_index_as,
           W_lin_src, b_lin_src, W_lin_agt, b_lin_agt,
           Wn_sa_0, Wr_sa_0, b_sa_0, Wn_as_0, Wr_as_0, b_as_0,
           Wn_sa_1, Wr_sa_1, b_sa_1, Wn_as_1, Wr_as_1, b_as_1,
           ln_g_src, ln_b_src, ln_g_agt, ln_b_agt,
           W_out, b_out, g):
  chunked = lambda v: v.astype(jnp.int32).reshape(NCHUNK, CH)
  src_sa = chunked(edge_index_sa[0])
  dst_sa = chunked(edge_index_sa[1])
  src_as = chunked(edge_index_as[0])
  dst_as = chunked(edge_index_as[1])

  def interleave(s2d, d2d):
    sd = jnp.stack([s2d, d2d], axis=1).reshape(2 * NCHUNK, CH)
    dmy_s = jnp.zeros((12, CH), jnp.int32)
    dmy_d = jnp.full((12, CH), N_PAD - CH, jnp.int32)
    dmy = jnp.stack([dmy_s, dmy_d], axis=1).reshape(24, CH)
    spare = jnp.zeros((4, CH), jnp.int32)
    return jnp.concatenate([sd, dmy, spare], axis=0)
  sd_sa = interleave(src_sa, dst_sa)
  sd_as = interleave(src_as, dst_as)

  row = lambda v: v.reshape(1, -1)
  hs0, ha0 = _kin(x_source, x_agent, W_lin_src, row(b_lin_src),
                  W_lin_agt, row(b_lin_agt))

  cnt_agt, cnt_src = _sc_cnt(dst_sa, dst_as)

  cat2 = lambda a, b: jnp.concatenate([a, b], axis=0)
  s_sa0, s_as0 = _sc_agg(hs0, ha0, sd_sa, sd_as)
  hs1, ha1 = _kupd(
      s_sa0, s_as0, hs0, ha0, cnt_agt, cnt_src,
      cat2(Wn_sa_0, Wr_sa_0), row(b_sa_0), cat2(Wn_as_0, Wr_as_0), row(b_as_0),
      row(ln_g_src), row(ln_b_src), row(ln_g_agt), row(ln_b_agt))

  s_sa1, s_as1 = _sc_agg(hs1, ha1, sd_sa, sd_as)
  return _kupd_out(
      s_sa1, s_as1, hs1, ha1, cnt_agt, cnt_src,
      cat2(Wn_sa_1, Wr_sa_1), row(b_sa_1), cat2(Wn_as_1, Wr_as_1), row(b_as_1),
      row(ln_g_src), row(ln_b_src), row(ln_g_agt), row(ln_b_agt),
      W_out, row(b_out), row(g))


# speedup vs baseline: 1.0277x; 1.0277x over previous
import jax
import jax.numpy as jnp
from jax import lax
from jax.experimental import pallas as pl
from jax.experimental.pallas import tpu as pltpu
from jax.experimental.pallas import tpu_sc as plsc

N = 10000
E = 320000
H = 128
EMB = 64

NS = 16
CH = 128
NCHUNK = E // CH
NPAIRS = 78
N_PAD = 10240
ROWS_PER_TILE = N_PAD // NS
WCH = 128
CNTW = 16


def _zero_fill(buf, nrows, width, value=0.0):
  def zrow(r, carry):
    for j in range(width // 16):
      buf[r, pl.ds(16 * j, 16)] = jnp.full((16,), value, jnp.float32)
    return carry
  lax.fori_loop(0, nrows, zrow, 0)


def _sc_agg_body(hs_hbm, ha_hbm, sd_sa, sd_as,
                 out_sa, out_as, acc, isd, rows0, rows1,
                 isem, gsem0, gsem1, ssem0, ssem1):
  c = lax.axis_index("c")
  s = lax.axis_index("s")
  r0row = 314 * s

  _zero_fill(rows0, WCH, H)
  base = s * ROWS_PER_TILE
  for k in range(ROWS_PER_TILE // WCH):
    pltpu.sync_copy(rows0, acc.at[pl.ds(base + k * WCH, WCH)])
  plsc.subcore_barrier()

  def run(table, sd_hbm):
    pltpu.async_copy(sd_hbm.at[pl.ds(r0row, 4)], isd.at[pl.ds(0, 4)], isem)

    def pair(q, carry):
      p4 = 4 * (q & 1)
      pn4 = 4 - p4
      sa_r = p4
      da_r = p4 + 1
      sb_r = p4 + 2
      db_r = p4 + 3
      pltpu.make_async_copy(sd_hbm.at[pl.ds(r0row, 4)], isd.at[pl.ds(0, 4)],
                            isem).wait()

      @pl.when(q > 0)
      def _():
        pltpu.make_async_copy(rows0, acc.at[isd.at[da_r]], ssem0).wait()
      pltpu.async_copy(table.at[isd.at[sa_r]], rows0, gsem0).wait()
      pltpu.async_copy(rows0, acc.at[isd.at[da_r]], ssem0, add=True)

      @pl.when(q > 0)
      def _():
        pltpu.make_async_copy(rows1, acc.at[isd.at[db_r]], ssem1).wait()
      pltpu.async_copy(sd_hbm.at[pl.ds(r0row + 4 * (q + 1), 4)],
                       isd.at[pl.ds(pn4, 4)], isem)
      pltpu.async_copy(table.at[isd.at[sb_r]], rows1, gsem1).wait()
      pltpu.async_copy(rows1, acc.at[isd.at[db_r]], ssem1, add=True)
      return carry

    lax.fori_loop(0, NPAIRS, pair, 0)

    pltpu.make_async_copy(sd_hbm.at[pl.ds(r0row, 4)], isd.at[pl.ds(0, 4)],
                          isem).wait()
    pltpu.make_async_copy(rows0, acc.at[isd.at[1]], ssem0).wait()
    pltpu.make_async_copy(rows1, acc.at[isd.at[1]], ssem1).wait()

    pltpu.async_copy(table.at[isd.at[0]], rows0, gsem0).wait()
    pltpu.sync_copy(rows0, acc.at[isd.at[1]], add=True)

  @pl.when(c == 0)
  def _():
    run(hs_hbm, sd_sa)

  @pl.when(c == 1)
  def _():
    run(ha_hbm, sd_as)

  plsc.subcore_barrier()

  def writeout(out_hbm):
    pltpu.sync_copy(acc.at[pl.ds(base, ROWS_PER_TILE)],
                    out_hbm.at[pl.ds(base, ROWS_PER_TILE)])

  @pl.when(c == 0)
  def _():
    writeout(out_sa)

  @pl.when(c == 1)
  def _():
    writeout(out_as)


_sc_agg = pl.kernel(
    _sc_agg_body,
    out_type=(jax.ShapeDtypeStruct((N_PAD, H), jnp.float32),
              jax.ShapeDtypeStruct((N_PAD, H), jnp.float32)),
    mesh=plsc.VectorSubcoreMesh(core_axis_name="c", subcore_axis_name="s"),
    compiler_params=pltpu.CompilerParams(use_tc_tiling_on_sc=False),
    scratch_types=[
        pltpu.VMEM_SHARED((N_PAD, H), jnp.float32),
        pltpu.VMEM((8, CH), jnp.int32),
        pltpu.VMEM((CH, H), jnp.float32),
        pltpu.VMEM((CH, H), jnp.float32),
        pltpu.SemaphoreType.DMA,
        pltpu.SemaphoreType.DMA,
        pltpu.SemaphoreType.DMA,
        pltpu.SemaphoreType.DMA,
        pltpu.SemaphoreType.DMA,
    ],
)


def _sc_cnt_body(dst_sa, dst_as, out_sa, out_as, acc, idst, ones, sem):
  c = lax.axis_index("c")
  s = lax.axis_index("s")
  q0 = 156 * s + jnp.minimum(s, 4)
  n_t = jnp.where(s < 4, 157, 156)
  q0c = jnp.minimum(q0, NCHUNK - 157)
  joff = q0 - q0c

  _zero_fill(ones, WCH, CNTW)
  base = s * ROWS_PER_TILE
  for k in range(ROWS_PER_TILE // WCH):
    pltpu.sync_copy(ones, acc.at[pl.ds(base + k * WCH, WCH)])
  plsc.subcore_barrier()
  _zero_fill(ones, WCH, CNTW, 1.0)

  def run(dst_hbm):
    pltpu.sync_copy(dst_hbm.at[pl.ds(q0c, 157)], idst)

    def chunk(j, carry):
      pltpu.sync_copy(ones, acc.at[idst.at[j + joff]], add=True)
      return carry
    lax.fori_loop(0, n_t, chunk, 0)

  @pl.when(c == 0)
  def _():
    run(dst_sa)

  @pl.when(c == 1)
  def _():
    run(dst_as)

  plsc.subcore_barrier()

  def writeout(out_hbm):
    pltpu.sync_copy(acc.at[pl.ds(base, ROWS_PER_TILE)],
                    out_hbm.at[pl.ds(base, ROWS_PER_TILE)])

  @pl.when(c == 0)
  def _():
    writeout(out_sa)

  @pl.when(c == 1)
  def _():
    writeout(out_as)


_sc_cnt = pl.kernel(
    _sc_cnt_body,
    out_type=(jax.ShapeDtypeStruct((N_PAD, CNTW), jnp.float32),
              jax.ShapeDtypeStruct((N_PAD, CNTW), jnp.float32)),
    mesh=plsc.VectorSubcoreMesh(core_axis_name="c", subcore_axis_name="s"),
    compiler_params=pltpu.CompilerParams(use_tc_tiling_on_sc=False),
    scratch_types=[
        pltpu.VMEM_SHARED((N_PAD, CNTW), jnp.float32),
        pltpu.VMEM((157, CH), jnp.int32),
        pltpu.VMEM((WCH, CNTW), jnp.float32),
        pltpu.SemaphoreType.DMA,
    ],
)


RBLK = 1000
GRID = N // RBLK


def _ln(x, gamma, beta):
  mu = jnp.mean(x, axis=-1, keepdims=True)
  var = jnp.mean((x - mu) ** 2, axis=-1, keepdims=True)
  return (x - mu) * lax.rsqrt(var + 1e-5) * gamma + beta


def _row_spec(d):
  return pl.BlockSpec((RBLK, d), lambda i: (i, 0))


def _full_spec(a, b):
  return pl.BlockSpec((a, b), lambda i: (0, 0))


def _kin_body(xs, xa, Wls, bls, Wla, bla, hs_out, ha_out):
  hs_out[...] = jnp.dot(xs[...], Wls[...],
                        preferred_element_type=jnp.float32) + bls[...]
  ha_out[...] = jnp.dot(xa[...], Wla[...],
                        preferred_element_type=jnp.float32) + bla[...]


def _kin(xs, xa, Wls, bls, Wla, bla):
  return pl.pallas_call(
      _kin_body,
      grid=(GRID,),
      in_specs=[_row_spec(H), _row_spec(H), _full_spec(H, H), _full_spec(1, H),
                _full_spec(H, H), _full_spec(1, H)],
      out_specs=(_row_spec(H), _row_spec(H)),
      out_shape=(jax.ShapeDtypeStruct((N, H), jnp.float32),
                 jax.ShapeDtypeStruct((N, H), jnp.float32)),
  )(xs, xa, Wls, bls, Wla, bla)


def _upd_one(s_agg, inv, h_dst, Wnr, b, ln_g, ln_b):
  cat = jnp.concatenate([s_agg * inv, h_dst], axis=1)
  new = jnp.dot(cat, Wnr[...], preferred_element_type=jnp.float32) + b[...]
  return _ln(jnp.maximum(new, 0.0), ln_g[...], ln_b[...]) + h_dst


def _kupd_body(ssa, sas, hs, ha, ca, cs, Wnr_sa, b_sa,
               Wnr_as, b_as, lgs, lbs, lga, lba, hs_o, ha_o):
  inv_a = 1.0 / jnp.maximum(ca[:, 0:1], 1.0)
  inv_s = 1.0 / jnp.maximum(cs[:, 0:1], 1.0)
  ha_o[...] = _upd_one(ssa[...], inv_a, ha[...], Wnr_sa, b_sa, lga, lba)
  hs_o[...] = _upd_one(sas[...], inv_s, hs[...], Wnr_as, b_as, lgs, lbs)


def _kupd(ssa, sas, hs, ha, ca, cs, Wnr_sa, b_sa,
          Wnr_as, b_as, lgs, lbs, lga, lba):
  wspec = _full_spec(2 * H, H)
  vspec = _full_spec(1, H)
  return pl.pallas_call(
      _kupd_body,
      grid=(GRID,),
      in_specs=[_row_spec(H), _row_spec(H), _row_spec(H), _row_spec(H),
                _row_spec(CNTW), _row_spec(CNTW),
                wspec, vspec, wspec, vspec,
                vspec, vspec, vspec, vspec],
      out_specs=(_row_spec(H), _row_spec(H)),
      out_shape=(jax.ShapeDtypeStruct((N, H), jnp.float32),
                 jax.ShapeDtypeStruct((N, H), jnp.float32)),
  )(ssa, sas, hs, ha, ca, cs, Wnr_sa, b_sa,
    Wnr_as, b_as, lgs, lbs, lga, lba)


def _out_proj(h, Wo, bo, g):
  o = jnp.dot(h, Wo[...], preferred_element_type=jnp.float32) + bo[...]
  nrm = jnp.sqrt(jnp.sum(o * o, axis=-1, keepdims=True))
  return o / jnp.maximum(nrm, 1e-12) * g[...]


def _kupd_out_body(ssa, sas, hs, ha, ca, cs, Wnr_sa, b_sa,
                   Wnr_as, b_as, lgs, lbs, lga, lba, Wo, bo, g,
                   os_o, oa_o):
  inv_a = 1.0 / jnp.maximum(ca[:, 0:1], 1.0)
  inv_s = 1.0 / jnp.maximum(cs[:, 0:1], 1.0)
  ha2 = _upd_one(ssa[...], inv_a, ha[...], Wnr_sa, b_sa, lga, lba)
  hs2 = _upd_one(sas[...], inv_s, hs[...], Wnr_as, b_as, lgs, lbs)
  os_o[...] = _out_proj(hs2, Wo, bo, g)
  oa_o[...] = _out_proj(ha2, Wo, bo, g)


def _kupd_out(ssa, sas, hs, ha, ca, cs, Wnr_sa, b_sa,
              Wnr_as, b_as, lgs, lbs, lga, lba, Wo, bo, g):
  wspec = _full_spec(2 * H, H)
  vspec = _full_spec(1, H)
  return pl.pallas_call(
      _kupd_out_body,
      grid=(GRID,),
      in_specs=[_row_spec(H), _row_spec(H), _row_spec(H), _row_spec(H),
                _row_spec(CNTW), _row_spec(CNTW),
                wspec, vspec, wspec, vspec,
                vspec, vspec, vspec, vspec,
                _full_spec(H, EMB), _full_spec(1, EMB), _full_spec(1, EMB)],
      out_specs=(_row_spec(EMB), _row_spec(EMB)),
      out_shape=(jax.ShapeDtypeStruct((N, EMB), jnp.float32),
                 jax.ShapeDtypeStruct((N, EMB), jnp.float32)),
  )(ssa, sas, hs, ha, ca, cs, Wnr_sa, b_sa,
    Wnr_as, b_as, lgs, lbs, lga, lba, Wo, bo, g)


def kernel(x_source, x_agent, edge_index_sa, edge_index_as,
           W_lin_src, b_lin_src, W_lin_agt, b_lin_agt,
           Wn_sa_0, Wr_sa_0, b_sa_0, Wn_as_0, Wr_as_0, b_as_0,
           Wn_sa_1, Wr_sa_1, b_sa_1, Wn_as_1, Wr_as_1, b_as_1,
           ln_g_src, ln_b_src, ln_g_agt, ln_b_agt,
           W_out, b_out, g):
  chunked = lambda v: v.astype(jnp.int32).reshape(NCHUNK, CH)
  src_sa = chunked(edge_index_sa[0])
  dst_sa = chunked(edge_index_sa[1])
  src_as = chunked(edge_index_as[0])
  dst_as = chunked(edge_index_as[1])

  def interleave(s2d, d2d):
    sd = jnp.stack([s2d, d2d], axis=1).reshape(2 * NCHUNK, CH)
    dmy_s = jnp.zeros((12, CH), jnp.int32)
    dmy_d = jnp.broadcast_to(N_PAD - CH + jnp.arange(CH, dtype=jnp.int32),
                             (12, CH))
    dmy = jnp.stack([dmy_s, dmy_d], axis=1).reshape(24, CH)
    spare = jnp.zeros((4, CH), jnp.int32)
    return jnp.concatenate([sd, dmy, spare], axis=0)
  sd_sa = interleave(src_sa, dst_sa)
  sd_as = interleave(src_as, dst_as)

  row = lambda v: v.reshape(1, -1)
  hs0, ha0 = _kin(x_source, x_agent, W_lin_src, row(b_lin_src),
                  W_lin_agt, row(b_lin_agt))

  cnt_agt, cnt_src = _sc_cnt(dst_sa, dst_as)

  cat2 = lambda a, b: jnp.concatenate([a, b], axis=0)
  s_sa0, s_as0 = _sc_agg(hs0, ha0, sd_sa, sd_as)
  hs1, ha1 = _kupd(
      s_sa0, s_as0, hs0, ha0, cnt_agt, cnt_src,
      cat2(Wn_sa_0, Wr_sa_0), row(b_sa_0), cat2(Wn_as_0, Wr_as_0), row(b_as_0),
      row(ln_g_src), row(ln_b_src), row(ln_g_agt), row(ln_b_agt))

  s_sa1, s_as1 = _sc_agg(hs1, ha1, sd_sa, sd_as)
  return _kupd_out(
      s_sa1, s_as1, hs1, ha1, cnt_agt, cnt_src,
      cat2(Wn_sa_1, Wr_sa_1), row(b_sa_1), cat2(Wn_as_1, Wr_as_1), row(b_as_1),
      row(ln_g_src), row(ln_b_src), row(ln_g_agt), row(ln_b_agt),
      W_out, row(b_out), row(g))


# speedup vs baseline: 1.2980x; 1.2630x over previous
import jax
import jax.numpy as jnp
from jax import lax
from jax.experimental import pallas as pl
from jax.experimental.pallas import tpu as pltpu
from jax.experimental.pallas import tpu_sc as plsc

N = 10000
E = 320000
H = 128
EMB = 64

NS = 16
CH = 128
NCHUNK = E // CH
NPAIRS = 78
N_PAD = 10240
ROWS_PER_TILE = N_PAD // NS
WCH = 128
CNTW = 16


def _zero_fill(buf, nrows, width, value=0.0):
  def zrow(r, carry):
    for j in range(width // 16):
      buf[r, pl.ds(16 * j, 16)] = jnp.full((16,), value, jnp.float32)
    return carry
  lax.fori_loop(0, nrows, zrow, 0)


def _sc_agg_body(hs_hbm, ha_hbm, src_sa, dst_sa, src_as, dst_as,
                 out_sa, out_as, acc, isrc, idst, rows0, rows1,
                 isem, gsem0, gsem1, ssem0, ssem1):
  c = lax.axis_index("c")
  s = lax.axis_index("s")
  q0 = 156 * s + jnp.minimum(s, 4)

  _zero_fill(rows0, WCH, H)
  base = s * ROWS_PER_TILE
  for k in range(ROWS_PER_TILE // WCH):
    pltpu.sync_copy(rows0, acc.at[pl.ds(base + k * WCH, WCH)])
  plsc.subcore_barrier()

  def run(table, src_hbm, dst_hbm):
    cp0 = pltpu.async_copy(src_hbm.at[pl.ds(q0, 2)], isrc.at[pl.ds(0, 2)], isem)
    cp1 = pltpu.async_copy(dst_hbm.at[pl.ds(q0, 2)], idst.at[pl.ds(0, 2)], isem)
    del cp0, cp1

    def pair(q, carry):
      p = 2 * (q & 1)
      pn = 2 - p
      a_row = p
      b_row = p + 1
      pltpu.make_async_copy(src_hbm.at[pl.ds(q0, 2)], isrc.at[pl.ds(0, 2)],
                            isem).wait()
      pltpu.make_async_copy(dst_hbm.at[pl.ds(q0, 2)], idst.at[pl.ds(0, 2)],
                            isem).wait()

      @pl.when(q > 0)
      def _():
        pltpu.make_async_copy(rows0, acc.at[idst.at[a_row]], ssem0).wait()
      pltpu.async_copy(table.at[isrc.at[a_row]], rows0, gsem0).wait()
      pltpu.async_copy(rows0, acc.at[idst.at[a_row]], ssem0, add=True)

      @pl.when(q > 0)
      def _():
        pltpu.make_async_copy(rows1, acc.at[idst.at[b_row]], ssem1).wait()
      ga_next = jnp.minimum(q0 + 2 * (q + 1), NCHUNK - 2)
      pltpu.async_copy(src_hbm.at[pl.ds(ga_next, 2)], isrc.at[pl.ds(pn, 2)],
                       isem)
      pltpu.async_copy(dst_hbm.at[pl.ds(ga_next, 2)], idst.at[pl.ds(pn, 2)],
                       isem)
      pltpu.async_copy(table.at[isrc.at[b_row]], rows1, gsem1).wait()
      pltpu.async_copy(rows1, acc.at[idst.at[b_row]], ssem1, add=True)
      return carry

    lax.fori_loop(0, NPAIRS, pair, 0)

    pltpu.make_async_copy(src_hbm.at[pl.ds(q0, 2)], isrc.at[pl.ds(0, 2)],
                          isem).wait()
    pltpu.make_async_copy(dst_hbm.at[pl.ds(q0, 2)], idst.at[pl.ds(0, 2)],
                          isem).wait()
    pltpu.make_async_copy(rows0, acc.at[idst.at[0]], ssem0).wait()
    pltpu.make_async_copy(rows1, acc.at[idst.at[1]], ssem1).wait()

    @pl.when(s < 4)
    def _():
      pltpu.async_copy(table.at[isrc.at[0]], rows0, gsem0).wait()
      pltpu.sync_copy(rows0, acc.at[idst.at[0]], add=True)

  @pl.when(c == 0)
  def _():
    run(hs_hbm, src_sa, dst_sa)

  @pl.when(c == 1)
  def _():
    run(ha_hbm, src_as, dst_as)

  plsc.subcore_barrier()

  def writeout(out_hbm):
    pltpu.sync_copy(acc.at[pl.ds(base, ROWS_PER_TILE)],
                    out_hbm.at[pl.ds(base, ROWS_PER_TILE)])

  @pl.when(c == 0)
  def _():
    writeout(out_sa)

  @pl.when(c == 1)
  def _():
    writeout(out_as)


_sc_agg = pl.kernel(
    _sc_agg_body,
    out_type=(jax.ShapeDtypeStruct((N_PAD, H), jnp.float32),
              jax.ShapeDtypeStruct((N_PAD, H), jnp.float32)),
    mesh=plsc.VectorSubcoreMesh(core_axis_name="c", subcore_axis_name="s"),
    compiler_params=pltpu.CompilerParams(use_tc_tiling_on_sc=False),
    scratch_types=[
        pltpu.VMEM_SHARED((N_PAD, H), jnp.float32),
        pltpu.VMEM((4, CH), jnp.int32),
        pltpu.VMEM((4, CH), jnp.int32),
        pltpu.VMEM((CH, H), jnp.float32),
        pltpu.VMEM((CH, H), jnp.float32),
        pltpu.SemaphoreType.DMA,
        pltpu.SemaphoreType.DMA,
        pltpu.SemaphoreType.DMA,
        pltpu.SemaphoreType.DMA,
        pltpu.SemaphoreType.DMA,
    ],
)


def _sc_cnt_body(dst_sa, dst_as, out_sa, out_as, acc, idst, ones, sem):
  c = lax.axis_index("c")
  s = lax.axis_index("s")
  q0 = 156 * s + jnp.minimum(s, 4)
  n_t = jnp.where(s < 4, 157, 156)
  q0c = jnp.minimum(q0, NCHUNK - 157)
  joff = q0 - q0c

  _zero_fill(ones, WCH, CNTW)
  base = s * ROWS_PER_TILE
  for k in range(ROWS_PER_TILE // WCH):
    pltpu.sync_copy(ones, acc.at[pl.ds(base + k * WCH, WCH)])
  plsc.subcore_barrier()
  _zero_fill(ones, WCH, CNTW, 1.0)

  def run(dst_hbm):
    pltpu.sync_copy(dst_hbm.at[pl.ds(q0c, 157)], idst)

    def chunk(j, carry):
      pltpu.async_copy(ones, acc.at[idst.at[j + joff]], sem, add=True)
      return carry
    lax.fori_loop(0, n_t, chunk, 0)

    def drain(j, carry):
      pltpu.make_async_copy(ones, acc.at[idst.at[0]], sem).wait()
      return carry
    lax.fori_loop(0, n_t, drain, 0)

  @pl.when(c == 0)
  def _():
    run(dst_sa)

  @pl.when(c == 1)
  def _():
    run(dst_as)

  plsc.subcore_barrier()

  def writeout(out_hbm):
    pltpu.sync_copy(acc.at[pl.ds(base, ROWS_PER_TILE)],
                    out_hbm.at[pl.ds(base, ROWS_PER_TILE)])

  @pl.when(c == 0)
  def _():
    writeout(out_sa)

  @pl.when(c == 1)
  def _():
    writeout(out_as)


_sc_cnt = pl.kernel(
    _sc_cnt_body,
    out_type=(jax.ShapeDtypeStruct((N_PAD, CNTW), jnp.float32),
              jax.ShapeDtypeStruct((N_PAD, CNTW), jnp.float32)),
    mesh=plsc.VectorSubcoreMesh(core_axis_name="c", subcore_axis_name="s"),
    compiler_params=pltpu.CompilerParams(use_tc_tiling_on_sc=False),
    scratch_types=[
        pltpu.VMEM_SHARED((N_PAD, CNTW), jnp.float32),
        pltpu.VMEM((157, CH), jnp.int32),
        pltpu.VMEM((WCH, CNTW), jnp.float32),
        pltpu.SemaphoreType.DMA,
    ],
)


RBLK = 1000
GRID = N // RBLK


def _ln(x, gamma, beta):
  mu = jnp.mean(x, axis=-1, keepdims=True)
  var = jnp.mean((x - mu) ** 2, axis=-1, keepdims=True)
  return (x - mu) * lax.rsqrt(var + 1e-5) * gamma + beta


def _row_spec(d):
  return pl.BlockSpec((RBLK, d), lambda i: (i, 0))


def _full_spec(a, b):
  return pl.BlockSpec((a, b), lambda i: (0, 0))


def _kin_body(xs, xa, Wls, bls, Wla, bla, hs_out, ha_out):
  hs_out[...] = jnp.dot(xs[...], Wls[...],
                        preferred_element_type=jnp.float32) + bls[...]
  ha_out[...] = jnp.dot(xa[...], Wla[...],
                        preferred_element_type=jnp.float32) + bla[...]


def _kin(xs, xa, Wls, bls, Wla, bla):
  return pl.pallas_call(
      _kin_body,
      grid=(GRID,),
      in_specs=[_row_spec(H), _row_spec(H), _full_spec(H, H), _full_spec(1, H),
                _full_spec(H, H), _full_spec(1, H)],
      out_specs=(_row_spec(H), _row_spec(H)),
      out_shape=(jax.ShapeDtypeStruct((N, H), jnp.float32),
                 jax.ShapeDtypeStruct((N, H), jnp.float32)),
  )(xs, xa, Wls, bls, Wla, bla)


def _upd_one(s_agg, inv, h_dst, Wnr, b, ln_g, ln_b):
  cat = jnp.concatenate([s_agg * inv, h_dst], axis=1)
  new = jnp.dot(cat, Wnr[...], preferred_element_type=jnp.float32) + b[...]
  return _ln(jnp.maximum(new, 0.0), ln_g[...], ln_b[...]) + h_dst


def _kupd_body(ssa, sas, hs, ha, ca, cs, Wnr_sa, b_sa,
               Wnr_as, b_as, lgs, lbs, lga, lba, hs_o, ha_o):
  inv_a = 1.0 / jnp.maximum(ca[:, 0:1], 1.0)
  inv_s = 1.0 / jnp.maximum(cs[:, 0:1], 1.0)
  ha_o[...] = _upd_one(ssa[...], inv_a, ha[...], Wnr_sa, b_sa, lga, lba)
  hs_o[...] = _upd_one(sas[...], inv_s, hs[...], Wnr_as, b_as, lgs, lbs)


def _kupd(ssa, sas, hs, ha, ca, cs, Wnr_sa, b_sa,
          Wnr_as, b_as, lgs, lbs, lga, lba):
  wspec = _full_spec(2 * H, H)
  vspec = _full_spec(1, H)
  return pl.pallas_call(
      _kupd_body,
      grid=(GRID,),
      in_specs=[_row_spec(H), _row_spec(H), _row_spec(H), _row_spec(H),
                _row_spec(CNTW), _row_spec(CNTW),
                wspec, vspec, wspec, vspec,
                vspec, vspec, vspec, vspec],
      out_specs=(_row_spec(H), _row_spec(H)),
      out_shape=(jax.ShapeDtypeStruct((N, H), jnp.float32),
                 jax.ShapeDtypeStruct((N, H), jnp.float32)),
  )(ssa, sas, hs, ha, ca, cs, Wnr_sa, b_sa,
    Wnr_as, b_as, lgs, lbs, lga, lba)


def _out_proj(h, Wo, bo, g):
  o = jnp.dot(h, Wo[...], preferred_element_type=jnp.float32) + bo[...]
  nrm = jnp.sqrt(jnp.sum(o * o, axis=-1, keepdims=True))
  return o / jnp.maximum(nrm, 1e-12) * g[...]


def _kupd_out_body(ssa, sas, hs, ha, ca, cs, Wnr_sa, b_sa,
                   Wnr_as, b_as, lgs, lbs, lga, lba, Wo, bo, g,
                   os_o, oa_o):
  inv_a = 1.0 / jnp.maximum(ca[:, 0:1], 1.0)
  inv_s = 1.0 / jnp.maximum(cs[:, 0:1], 1.0)
  ha2 = _upd_one(ssa[...], inv_a, ha[...], Wnr_sa, b_sa, lga, lba)
  hs2 = _upd_one(sas[...], inv_s, hs[...], Wnr_as, b_as, lgs, lbs)
  os_o[...] = _out_proj(hs2, Wo, bo, g)
  oa_o[...] = _out_proj(ha2, Wo, bo, g)


def _kupd_out(ssa, sas, hs, ha, ca, cs, Wnr_sa, b_sa,
              Wnr_as, b_as, lgs, lbs, lga, lba, Wo, bo, g):
  wspec = _full_spec(2 * H, H)
  vspec = _full_spec(1, H)
  return pl.pallas_call(
      _kupd_out_body,
      grid=(GRID,),
      in_specs=[_row_spec(H), _row_spec(H), _row_spec(H), _row_spec(H),
                _row_spec(CNTW), _row_spec(CNTW),
                wspec, vspec, wspec, vspec,
                vspec, vspec, vspec, vspec,
                _full_spec(H, EMB), _full_spec(1, EMB), _full_spec(1, EMB)],
      out_specs=(_row_spec(EMB), _row_spec(EMB)),
      out_shape=(jax.ShapeDtypeStruct((N, EMB), jnp.float32),
                 jax.ShapeDtypeStruct((N, EMB), jnp.float32)),
  )(ssa, sas, hs, ha, ca, cs, Wnr_sa, b_sa,
    Wnr_as, b_as, lgs, lbs, lga, lba, Wo, bo, g)


def kernel(x_source, x_agent, edge_index_sa, edge_index_as,
           W_lin_src, b_lin_src, W_lin_agt, b_lin_agt,
           Wn_sa_0, Wr_sa_0, b_sa_0, Wn_as_0, Wr_as_0, b_as_0,
           Wn_sa_1, Wr_sa_1, b_sa_1, Wn_as_1, Wr_as_1, b_as_1,
           ln_g_src, ln_b_src, ln_g_agt, ln_b_agt,
           W_out, b_out, g):
  chunked = lambda v: v.astype(jnp.int32).reshape(NCHUNK, CH)
  src_sa = chunked(edge_index_sa[0])
  dst_sa = chunked(edge_index_sa[1])
  src_as = chunked(edge_index_as[0])
  dst_as = chunked(edge_index_as[1])

  row = lambda v: v.reshape(1, -1)
  hs0, ha0 = _kin(x_source, x_agent, W_lin_src, row(b_lin_src),
                  W_lin_agt, row(b_lin_agt))

  cnt_agt, cnt_src = _sc_cnt(dst_sa, dst_as)

  cat2 = lambda a, b: jnp.concatenate([a, b], axis=0)
  s_sa0, s_as0 = _sc_agg(hs0, ha0, src_sa, dst_sa, src_as, dst_as)
  hs1, ha1 = _kupd(
      s_sa0, s_as0, hs0, ha0, cnt_agt, cnt_src,
      cat2(Wn_sa_0, Wr_sa_0), row(b_sa_0), cat2(Wn_as_0, Wr_as_0), row(b_as_0),
      row(ln_g_src), row(ln_b_src), row(ln_g_agt), row(ln_b_agt))

  s_sa1, s_as1 = _sc_agg(hs1, ha1, src_sa, dst_sa, src_as, dst_as)
  return _kupd_out(
      s_sa1, s_as1, hs1, ha1, cnt_agt, cnt_src,
      cat2(Wn_sa_1, Wr_sa_1), row(b_sa_1), cat2(Wn_as_1, Wr_as_1), row(b_as_1),
      row(ln_g_src), row(ln_b_src), row(ln_g_agt), row(ln_b_agt),
      W_out, row(b_out), row(g))


# speedup vs baseline: 1.3052x; 1.0056x over previous
import jax
import jax.numpy as jnp
from jax import lax
from jax.experimental import pallas as pl
from jax.experimental.pallas import tpu as pltpu
from jax.experimental.pallas import tpu_sc as plsc

N = 10000
E = 320000
H = 128
EMB = 64

NS = 16
CH = 128
NCHUNK = E // CH
NPAIRS = 78
N_PAD = 10240
ROWS_PER_TILE = N_PAD // NS
WCH = 128
CNTW = 16


def _zero_fill(buf, nrows, width, value=0.0):
  def zrow(r, carry):
    for j in range(width // 16):
      buf[r, pl.ds(16 * j, 16)] = jnp.full((16,), value, jnp.float32)
    return carry
  lax.fori_loop(0, nrows, zrow, 0)


def _sc_agg_body(hs_hbm, ha_hbm, src_sa, dst_sa, src_as, dst_as,
                 out_sa, out_as, acc, isrc, idst, rows0, rows1,
                 isem, gsem0, gsem1, ssem0, ssem1):
  c = lax.axis_index("c")
  s = lax.axis_index("s")
  q0 = 156 * s + jnp.minimum(s, 4)

  _zero_fill(rows0, WCH, H)
  base = s * ROWS_PER_TILE
  for k in range(ROWS_PER_TILE // WCH):
    pltpu.sync_copy(rows0, acc.at[pl.ds(base + k * WCH, WCH)])
  plsc.subcore_barrier()

  def run(table, src_hbm, dst_hbm):
    cp0 = pltpu.async_copy(src_hbm.at[pl.ds(q0, 2)], isrc.at[pl.ds(0, 2)], isem)
    cp1 = pltpu.async_copy(dst_hbm.at[pl.ds(q0, 2)], idst.at[pl.ds(0, 2)], isem)
    del cp0, cp1

    def pair(q, carry):
      p = 2 * (q & 1)
      pn = 2 - p
      a_row = p
      b_row = p + 1
      pltpu.make_async_copy(src_hbm.at[pl.ds(q0, 2)], isrc.at[pl.ds(0, 2)],
                            isem).wait()
      pltpu.make_async_copy(dst_hbm.at[pl.ds(q0, 2)], idst.at[pl.ds(0, 2)],
                            isem).wait()

      @pl.when(q > 0)
      def _():
        pltpu.make_async_copy(rows0, acc.at[idst.at[a_row]], ssem0).wait()
      pltpu.async_copy(table.at[isrc.at[a_row]], rows0, gsem0).wait()
      pltpu.async_copy(rows0, acc.at[idst.at[a_row]], ssem0, add=True)

      @pl.when(q > 0)
      def _():
        pltpu.make_async_copy(rows1, acc.at[idst.at[b_row]], ssem1).wait()
      ga_next = jnp.minimum(q0 + 2 * (q + 1), NCHUNK - 2)
      pltpu.async_copy(src_hbm.at[pl.ds(ga_next, 2)], isrc.at[pl.ds(pn, 2)],
                       isem)
      pltpu.async_copy(dst_hbm.at[pl.ds(ga_next, 2)], idst.at[pl.ds(pn, 2)],
                       isem)
      pltpu.async_copy(table.at[isrc.at[b_row]], rows1, gsem1).wait()
      pltpu.async_copy(rows1, acc.at[idst.at[b_row]], ssem1, add=True)
      return carry

    lax.fori_loop(0, NPAIRS, pair, 0)

    pltpu.make_async_copy(src_hbm.at[pl.ds(q0, 2)], isrc.at[pl.ds(0, 2)],
                          isem).wait()
    pltpu.make_async_copy(dst_hbm.at[pl.ds(q0, 2)], idst.at[pl.ds(0, 2)],
                          isem).wait()
    pltpu.make_async_copy(rows0, acc.at[idst.at[0]], ssem0).wait()
    pltpu.make_async_copy(rows1, acc.at[idst.at[1]], ssem1).wait()

    @pl.when(s < 4)
    def _():
      pltpu.async_copy(table.at[isrc.at[0]], rows0, gsem0).wait()
      pltpu.sync_copy(rows0, acc.at[idst.at[0]], add=True)

  @pl.when(c == 0)
  def _():
    run(hs_hbm, src_sa, dst_sa)

  @pl.when(c == 1)
  def _():
    run(ha_hbm, src_as, dst_as)

  plsc.subcore_barrier()

  def writeout(out_hbm):
    pltpu.sync_copy(acc.at[pl.ds(base, ROWS_PER_TILE)],
                    out_hbm.at[pl.ds(base, ROWS_PER_TILE)])

  @pl.when(c == 0)
  def _():
    writeout(out_sa)

  @pl.when(c == 1)
  def _():
    writeout(out_as)


_sc_agg = pl.kernel(
    _sc_agg_body,
    out_type=(jax.ShapeDtypeStruct((N_PAD, H), jnp.float32),
              jax.ShapeDtypeStruct((N_PAD, H), jnp.float32)),
    mesh=plsc.VectorSubcoreMesh(core_axis_name="c", subcore_axis_name="s"),
    compiler_params=pltpu.CompilerParams(use_tc_tiling_on_sc=False),
    scratch_types=[
        pltpu.VMEM_SHARED((N_PAD, H), jnp.float32),
        pltpu.VMEM((4, CH), jnp.int32),
        pltpu.VMEM((4, CH), jnp.int32),
        pltpu.VMEM((CH, H), jnp.float32),
        pltpu.VMEM((CH, H), jnp.float32),
        pltpu.SemaphoreType.DMA,
        pltpu.SemaphoreType.DMA,
        pltpu.SemaphoreType.DMA,
        pltpu.SemaphoreType.DMA,
        pltpu.SemaphoreType.DMA,
    ],
)


def _sc_cnt_body(dst_sa, dst_as, out_sa, out_as, acc, idst, ones, sem):
  c = lax.axis_index("c")
  s = lax.axis_index("s")
  q0 = 156 * s + jnp.minimum(s, 4)
  n_t = jnp.where(s < 4, 157, 156)
  q0c = jnp.minimum(q0, NCHUNK - 157)
  joff = q0 - q0c

  _zero_fill(ones, WCH, CNTW)
  base = s * ROWS_PER_TILE
  for k in range(ROWS_PER_TILE // WCH):
    pltpu.sync_copy(ones, acc.at[pl.ds(base + k * WCH, WCH)])
  plsc.subcore_barrier()
  _zero_fill(ones, WCH, CNTW, 1.0)

  def run(dst_hbm):
    pltpu.sync_copy(dst_hbm.at[pl.ds(q0c, 157)], idst)

    def chunk(j, carry):
      pltpu.sync_copy(ones, acc.at[idst.at[j + joff]], add=True)
      return carry
    lax.fori_loop(0, n_t, chunk, 0)

  @pl.when(c == 0)
  def _():
    run(dst_sa)

  @pl.when(c == 1)
  def _():
    run(dst_as)

  plsc.subcore_barrier()

  def writeout(out_hbm):
    pltpu.sync_copy(acc.at[pl.ds(base, ROWS_PER_TILE)],
                    out_hbm.at[pl.ds(base, ROWS_PER_TILE)])

  @pl.when(c == 0)
  def _():
    writeout(out_sa)

  @pl.when(c == 1)
  def _():
    writeout(out_as)


_sc_cnt = pl.kernel(
    _sc_cnt_body,
    out_type=(jax.ShapeDtypeStruct((N_PAD, CNTW), jnp.float32),
              jax.ShapeDtypeStruct((N_PAD, CNTW), jnp.float32)),
    mesh=plsc.VectorSubcoreMesh(core_axis_name="c", subcore_axis_name="s"),
    compiler_params=pltpu.CompilerParams(use_tc_tiling_on_sc=False),
    scratch_types=[
        pltpu.VMEM_SHARED((N_PAD, CNTW), jnp.float32),
        pltpu.VMEM((157, CH), jnp.int32),
        pltpu.VMEM((WCH, CNTW), jnp.float32),
        pltpu.SemaphoreType.DMA,
    ],
)


RBLK = 2000
GRID = N // RBLK


def _ln(x, gamma, beta):
  mu = jnp.mean(x, axis=-1, keepdims=True)
  var = jnp.mean((x - mu) ** 2, axis=-1, keepdims=True)
  return (x - mu) * lax.rsqrt(var + 1e-5) * gamma + beta


def _row_spec(d):
  return pl.BlockSpec((RBLK, d), lambda i: (i, 0))


def _full_spec(a, b):
  return pl.BlockSpec((a, b), lambda i: (0, 0))


def _kin_body(xs, xa, Wls, bls, Wla, bla, hs_out, ha_out):
  hs_out[...] = jnp.dot(xs[...], Wls[...],
                        preferred_element_type=jnp.float32) + bls[...]
  ha_out[...] = jnp.dot(xa[...], Wla[...],
                        preferred_element_type=jnp.float32) + bla[...]


def _kin(xs, xa, Wls, bls, Wla, bla):
  return pl.pallas_call(
      _kin_body,
      grid=(GRID,),
      in_specs=[_row_spec(H), _row_spec(H), _full_spec(H, H), _full_spec(1, H),
                _full_spec(H, H), _full_spec(1, H)],
      out_specs=(_row_spec(H), _row_spec(H)),
      out_shape=(jax.ShapeDtypeStruct((N, H), jnp.float32),
                 jax.ShapeDtypeStruct((N, H), jnp.float32)),
  )(xs, xa, Wls, bls, Wla, bla)


def _upd_one(s_agg, inv, h_dst, Wnr, b, ln_g, ln_b):
  cat = jnp.concatenate([s_agg * inv, h_dst], axis=1)
  new = jnp.dot(cat, Wnr[...], preferred_element_type=jnp.float32) + b[...]
  return _ln(jnp.maximum(new, 0.0), ln_g[...], ln_b[...]) + h_dst


def _kupd_body(ssa, sas, hs, ha, ca, cs, Wnr_sa, b_sa,
               Wnr_as, b_as, lgs, lbs, lga, lba, hs_o, ha_o):
  inv_a = 1.0 / jnp.maximum(ca[:, 0:1], 1.0)
  inv_s = 1.0 / jnp.maximum(cs[:, 0:1], 1.0)
  ha_o[...] = _upd_one(ssa[...], inv_a, ha[...], Wnr_sa, b_sa, lga, lba)
  hs_o[...] = _upd_one(sas[...], inv_s, hs[...], Wnr_as, b_as, lgs, lbs)


def _kupd(ssa, sas, hs, ha, ca, cs, Wnr_sa, b_sa,
          Wnr_as, b_as, lgs, lbs, lga, lba):
  wspec = _full_spec(2 * H, H)
  vspec = _full_spec(1, H)
  return pl.pallas_call(
      _kupd_body,
      grid=(GRID,),
      in_specs=[_row_spec(H), _row_spec(H), _row_spec(H), _row_spec(H),
                _row_spec(CNTW), _row_spec(CNTW),
                wspec, vspec, wspec, vspec,
                vspec, vspec, vspec, vspec],
      out_specs=(_row_spec(H), _row_spec(H)),
      out_shape=(jax.ShapeDtypeStruct((N, H), jnp.float32),
                 jax.ShapeDtypeStruct((N, H), jnp.float32)),
  )(ssa, sas, hs, ha, ca, cs, Wnr_sa, b_sa,
    Wnr_as, b_as, lgs, lbs, lga, lba)


def _out_proj(h, Wo, bo, g):
  o = jnp.dot(h, Wo[...], preferred_element_type=jnp.float32) + bo[...]
  nrm = jnp.sqrt(jnp.sum(o * o, axis=-1, keepdims=True))
  return o / jnp.maximum(nrm, 1e-12) * g[...]


def _kupd_out_body(ssa, sas, hs, ha, ca, cs, Wnr_sa, b_sa,
                   Wnr_as, b_as, lgs, lbs, lga, lba, Wo, bo, g,
                   os_o, oa_o):
  inv_a = 1.0 / jnp.maximum(ca[:, 0:1], 1.0)
  inv_s = 1.0 / jnp.maximum(cs[:, 0:1], 1.0)
  ha2 = _upd_one(ssa[...], inv_a, ha[...], Wnr_sa, b_sa, lga, lba)
  hs2 = _upd_one(sas[...], inv_s, hs[...], Wnr_as, b_as, lgs, lbs)
  os_o[...] = _out_proj(hs2, Wo, bo, g)
  oa_o[...] = _out_proj(ha2, Wo, bo, g)


def _kupd_out(ssa, sas, hs, ha, ca, cs, Wnr_sa, b_sa,
              Wnr_as, b_as, lgs, lbs, lga, lba, Wo, bo, g):
  wspec = _full_spec(2 * H, H)
  vspec = _full_spec(1, H)
  return pl.pallas_call(
      _kupd_out_body,
      grid=(GRID,),
      in_specs=[_row_spec(H), _row_spec(H), _row_spec(H), _row_spec(H),
                _row_spec(CNTW), _row_spec(CNTW),
                wspec, vspec, wspec, vspec,
                vspec, vspec, vspec, vspec,
                _full_spec(H, EMB), _full_spec(1, EMB), _full_spec(1, EMB)],
      out_specs=(_row_spec(EMB), _row_spec(EMB)),
      out_shape=(jax.ShapeDtypeStruct((N, EMB), jnp.float32),
                 jax.ShapeDtypeStruct((N, EMB), jnp.float32)),
  )(ssa, sas, hs, ha, ca, cs, Wnr_sa, b_sa,
    Wnr_as, b_as, lgs, lbs, lga, lba, Wo, bo, g)


def kernel(x_source, x_agent, edge_index_sa, edge_index_as,
           W_lin_src, b_lin_src, W_lin_agt, b_lin_agt,
           Wn_sa_0, Wr_sa_0, b_sa_0, Wn_as_0, Wr_as_0, b_as_0,
           Wn_sa_1, Wr_sa_1, b_sa_1, Wn_as_1, Wr_as_1, b_as_1,
           ln_g_src, ln_b_src, ln_g_agt, ln_b_agt,
           W_out, b_out, g):
  chunked = lambda v: v.astype(jnp.int32).reshape(NCHUNK, CH)
  src_sa = chunked(edge_index_sa[0])
  dst_sa = chunked(edge_index_sa[1])
  src_as = chunked(edge_index_as[0])
  dst_as = chunked(edge_index_as[1])

  row = lambda v: v.reshape(1, -1)
  hs0, ha0 = _kin(x_source, x_agent, W_lin_src, row(b_lin_src),
                  W_lin_agt, row(b_lin_agt))

  cnt_agt, cnt_src = _sc_cnt(dst_sa, dst_as)

  cat2 = lambda a, b: jnp.concatenate([a, b], axis=0)
  s_sa0, s_as0 = _sc_agg(hs0, ha0, src_sa, dst_sa, src_as, dst_as)
  hs1, ha1 = _kupd(
      s_sa0, s_as0, hs0, ha0, cnt_agt, cnt_src,
      cat2(Wn_sa_0, Wr_sa_0), row(b_sa_0), cat2(Wn_as_0, Wr_as_0), row(b_as_0),
      row(ln_g_src), row(ln_b_src), row(ln_g_agt), row(ln_b_agt))

  s_sa1, s_as1 = _sc_agg(hs1, ha1, src_sa, dst_sa, src_as, dst_as)
  return _kupd_out(
      s_sa1, s_as1, hs1, ha1, cnt_agt, cnt_src,
      cat2(Wn_sa_1, Wr_sa_1), row(b_sa_1), cat2(Wn_as_1, Wr_as_1), row(b_as_1),
      row(ln_g_src), row(ln_b_src), row(ln_g_agt), row(ln_b_agt),
      W_out, row(b_out), row(g))
